# baseline (speedup 1.0000x reference)
import functools
import math

import jax
import jax.numpy as jnp
from jax import lax
from jax.experimental import pallas as pl
from jax.experimental.pallas import tpu as pltpu
from jax.experimental.pallas import tpu_sc as plsc

D_MODEL = 64
SCALE = math.sqrt(D_MODEL)

IDXW = 128
CHUNK = 256
KSUB = CHUNK // IDXW
NBUF = 6
LAG = 3


@functools.lru_cache(maxsize=None)
def _make_gather(b, v, d):
    info = plsc.get_sparse_core_info()
    nc, ns = info.num_cores, info.num_subcores
    nw = nc * ns
    assert b % (nw * CHUNK) == 0 and (CHUNK * d) % 256 == 0
    b_per_w = b // nw
    g_total = b_per_w // CHUNK
    assert g_total > NBUF
    idx_rows_per_w = b_per_w // IDXW
    grows = IDXW * d // 128
    crows = CHUNK * d // 128

    mesh = plsc.VectorSubcoreMesh(core_axis_name="c", subcore_axis_name="s")

    @functools.partial(
        pl.kernel,
        mesh=mesh,
        compiler_params=pltpu.CompilerParams(use_tc_tiling_on_sc=False),
        out_type=jax.ShapeDtypeStruct((b, d), jnp.float32),
        scratch_types=[
            pltpu.VMEM((idx_rows_per_w, IDXW), jnp.int32),
            pltpu.VMEM((NBUF, CHUNK, d), jnp.float32),
        ]
        + [pltpu.SemaphoreType.DMA] * (2 * NBUF),
    )
    def gather_kernel(idx_hbm, tab_hbm, out_hbm, idx_v, rows_v, *sems):
        gsems = sems[:NBUF]
        osems = sems[NBUF:]
        wid = lax.axis_index("s") * nc + lax.axis_index("c")
        row_base = wid * b_per_w
        idx_row_base = wid * idx_rows_per_w

        pltpu.sync_copy(
            idx_hbm.at[pl.ds(idx_row_base, idx_rows_per_w)], idx_v
        )

        def gather_descs(g):
            bslot = g % NBUF
            return [
                (
                    tab_hbm.at[idx_v.at[g * KSUB + j]],
                    rows_v.at[bslot, pl.ds(j * IDXW, IDXW)],
                    gsems[bslot],
                )
                for j in range(KSUB)
            ]

        def write_desc(g):
            bslot = g % NBUF
            return (
                rows_v.at[bslot],
                out_hbm.at[pl.ds(row_base + g * CHUNK, CHUNK)],
                osems[bslot],
            )

        for g in range(g_total + LAG):
            if g < g_total:
                if g >= NBUF:
                    pltpu.make_async_copy(*write_desc(g - NBUF)).wait()
                for desc in gather_descs(g):
                    pltpu.async_copy(*desc)
            gp = g - LAG
            if gp >= 0:
                for desc in gather_descs(gp):
                    pltpu.make_async_copy(*desc).wait()
                pltpu.async_copy(*write_desc(gp))

        for g in range(g_total + LAG - NBUF, g_total):
            pltpu.make_async_copy(*write_desc(g)).wait()

    return gather_kernel


def _finish_body(i_ref, o_ref):
    x2 = i_ref[...] * SCALE
    cat = jnp.concatenate([x2[:, :64], x2[:, 64:]], axis=0)
    o_ref[...] = cat.reshape(o_ref.shape)


S_BLK = 64


@functools.lru_cache(maxsize=None)
def _make_finish(s, p, d, s_blk):
    assert s % s_blk == 0 and (s_blk * p * d) % 128 == 0
    in_rows = s_blk * p * d // 128
    return pl.pallas_call(
        _finish_body,
        grid=(s // s_blk,),
        in_specs=[pl.BlockSpec((in_rows, 128), lambda i: (i, 0))],
        out_specs=pl.BlockSpec((s_blk, p, d), lambda i: (i, 0, 0)),
        out_shape=jax.ShapeDtypeStruct((s, p, d), jnp.float32),
    )


def kernel(x, table):
    v, d = table.shape
    s, p = x.shape
    b = x.size
    nblk = s // S_BLK
    half = S_BLK * p // 2
    xp = (
        x.reshape(nblk, 2, half)
        .swapaxes(1, 2)
        .reshape(b // IDXW, IDXW)
        .astype(jnp.int32)
    )
    interm = _make_gather(b, v, d)(xp, table)
    interm2 = interm.reshape(b * d // 128, 128)
    return _make_finish(s, p, d, S_BLK)(interm2)
